# speedup vs baseline: 1.3871x; 1.3871x over previous
import functools

import jax
import jax.numpy as jnp
from jax import lax
from jax.experimental import pallas as pl
from jax.experimental.pallas import tpu as pltpu
from jax.experimental.pallas import tpu_sc as plsc


def _gather_kernel(B, b_per_w, NC):
    mesh = plsc.VectorSubcoreMesh(core_axis_name="c", subcore_axis_name="s")

    @functools.partial(
        pl.kernel,
        mesh=mesh,
        out_type=jax.ShapeDtypeStruct((B,), jnp.float32),
        scratch_types=[
            pltpu.VMEM((b_per_w,), jnp.int32),
            pltpu.VMEM((b_per_w,), jnp.float32),
            pltpu.SemaphoreType.DMA,
        ],
    )
    def k(table_hbm, idx_hbm, out_hbm, idx_v, vals_v, sem):
        wid = lax.axis_index("s") * NC + lax.axis_index("c")
        base = wid * b_per_w
        pltpu.sync_copy(idx_hbm.at[pl.ds(base, b_per_w)], idx_v)
        pltpu.async_copy(table_hbm.at[idx_v], vals_v, sem).wait()
        pltpu.sync_copy(vals_v, out_hbm.at[pl.ds(base, b_per_w)])

    return k


def kernel(fit_X_col, donors_idx):
    B0, K = donors_idx.shape
    B = B0 * K
    info = plsc.get_sparse_core_info()
    NW = info.num_cores * info.num_subcores
    assert B % (8 * NW) == 0
    b_per_w = B // NW
    idx_flat = donors_idx.astype(jnp.int32).reshape(B)
    out = _gather_kernel(B, b_per_w, info.num_cores)(fit_X_col, idx_flat)
    return out.reshape(B0, K)


# speedup vs baseline: 1.6056x; 1.1575x over previous
import functools

import jax
import jax.numpy as jnp
from jax import lax
from jax.experimental import pallas as pl
from jax.experimental.pallas import tpu as pltpu
from jax.experimental.pallas import tpu_sc as plsc


def _gather_kernel(V, B, b_per_w, NC, NS):
    mesh = plsc.VectorSubcoreMesh(core_axis_name="c", subcore_axis_name="s")
    PS = 10000
    assert V % PS == 0 and PS % 8 == 0
    NP = V // PS
    max_i = (NP + NS - 1) // NS

    @functools.partial(
        pl.kernel,
        mesh=mesh,
        out_type=jax.ShapeDtypeStruct((B,), jnp.float32),
        scratch_types=[
            pltpu.VMEM_SHARED((V,), jnp.float32),
            pltpu.VMEM((PS,), jnp.float32),
            pltpu.VMEM((b_per_w,), jnp.int32),
            pltpu.VMEM((b_per_w,), jnp.float32),
            pltpu.SemaphoreType.DMA,
        ],
    )
    def k(table_hbm, idx_hbm, out_hbm, shared, stage_v, idx_v, vals_v, sem):
        c = lax.axis_index("c")
        s = lax.axis_index("s")
        wid = s * NC + c
        base = wid * b_per_w
        pltpu.sync_copy(idx_hbm.at[pl.ds(base, b_per_w)], idx_v)
        for i in range(max_i):
            p = i * NS + s

            @pl.when(p < NP)
            def _():
                off = p * PS
                pltpu.sync_copy(table_hbm.at[pl.ds(off, PS)], stage_v)
                pltpu.sync_copy(stage_v, shared.at[pl.ds(off, PS)])

        plsc.subcore_barrier()
        pltpu.async_copy(shared.at[idx_v], vals_v, sem).wait()
        pltpu.sync_copy(vals_v, out_hbm.at[pl.ds(base, b_per_w)])

    return k


def kernel(fit_X_col, donors_idx):
    B0, K = donors_idx.shape
    B = B0 * K
    V = fit_X_col.shape[0]
    info = plsc.get_sparse_core_info()
    NC, NS = info.num_cores, info.num_subcores
    NW = NC * NS
    assert B % (8 * NW) == 0
    b_per_w = B // NW
    idx_flat = donors_idx.astype(jnp.int32).reshape(B)
    out = _gather_kernel(V, B, b_per_w, NC, NS)(fit_X_col, idx_flat)
    return out.reshape(B0, K)


# speedup vs baseline: 2.5389x; 1.5813x over previous
import functools

import jax
import jax.numpy as jnp
from jax import lax
from jax.experimental import pallas as pl
from jax.experimental.pallas import tpu as pltpu
from jax.experimental.pallas import tpu_sc as plsc


def _gather_kernel(V, B, b_per_w, NC, NS):
    mesh = plsc.VectorSubcoreMesh(core_axis_name="c", subcore_axis_name="s")
    PS = 10000
    assert V % PS == 0 and PS % 8 == 0
    NP = V // PS
    max_i = (NP + NS - 1) // NS

    @functools.partial(
        pl.kernel,
        mesh=mesh,
        out_type=jax.ShapeDtypeStruct((B,), jnp.float32),
        scratch_types=[
            pltpu.VMEM_SHARED((V,), jnp.float32),
            pltpu.VMEM((PS,), jnp.float32),
            pltpu.VMEM((b_per_w,), jnp.int32),
            pltpu.VMEM((b_per_w,), jnp.float32),
            pltpu.SemaphoreType.DMA,
        ],
    )
    def k(table_hbm, idx_hbm, out_hbm, shared, stage_v, idx_v, vals_v, sem):
        c = lax.axis_index("c")
        s = lax.axis_index("s")
        wid = s * NC + c
        base = wid * b_per_w
        pltpu.sync_copy(idx_hbm.at[pl.ds(base, b_per_w)], idx_v)
        for i in range(max_i):
            p = i * NS + s

            @pl.when(p < NP)
            def _():
                off = p * PS
                pltpu.sync_copy(table_hbm.at[pl.ds(off, PS)], stage_v)
                pltpu.sync_copy(stage_v, shared.at[pl.ds(off, PS)])

        plsc.subcore_barrier()
        pltpu.async_copy(shared.at[idx_v], vals_v, sem).wait()
        pltpu.sync_copy(vals_v, out_hbm.at[pl.ds(base, b_per_w)])

    return k


def kernel(fit_X_col, donors_idx):
    B0, K = donors_idx.shape
    B = B0 * K
    V = fit_X_col.shape[0]
    info = plsc.get_sparse_core_info()
    NC, NS = info.num_cores, info.num_subcores
    NW = NC * NS
    assert B % (8 * NW) == 0
    b_per_w = B // NW
    idx_flat = donors_idx.astype(jnp.int32).T.reshape(B)
    out = _gather_kernel(V, B, b_per_w, NC, NS)(fit_X_col, idx_flat)
    return out.reshape(K, B0).T
